# Initial kernel scaffold; baseline (speedup 1.0000x reference)
import jax
import jax.numpy as jnp
from jax.experimental import pallas as pl


def kernel(mask_feature, feature, edge_index, edge_type, W1, b1, Wg1, bg1, Wg2, bg2, Wout, bout):
    raise NotImplementedError("write your pallas kernel here")


# speedup vs baseline: 6.7478x; 6.7478x over previous
import functools

import jax
import jax.numpy as jnp
from jax import lax
from jax.experimental import pallas as pl
from jax.experimental.pallas import tpu as pltpu
from jax.experimental.pallas import tpu_sc as plsc

N = 100000
E = 1600000
D_IN = 16
H = 128
OUT = 2

NC, NS = 2, 16
N_ACC = 102400
H0 = N_ACC // NC
ACC_ROWS = 51712
ACC_W = ACC_ROWS * 4
DUMP = H0
E_PAD = 1638400
EROWS = E_PAD // 128
RPT = EROWS // NS
CH = 100
NCHUNK = RPT // CH
G = 4
STRW = ACC_W // NS
WIDTH = 4
BN = 2048
BNC = 2000

_F32 = jnp.float32


def _sc_body(cols, edges, table, zt, part,
             src_b, dst_b, rows, acc, gsem, ssem, isem):
    c = lax.axis_index("c")
    s = lax.axis_index("s")
    base_w = s * STRW

    @pl.loop(0, STRW // 128)
    def _(k):
        pltpu.sync_copy(zt, acc.at[pl.ds(base_w + k * 128, 128)])

    plsc.subcore_barrier()

    for ch in range(NCHUNK):
        rb = s * RPT + ch * CH
        i1 = pltpu.async_copy(edges.at[0, pl.ds(rb, CH)], src_b, isem)
        i2 = pltpu.async_copy(edges.at[1 + c, pl.ds(rb, CH)], dst_b, isem)
        i1.wait()
        i2.wait()

        @pl.loop(0, CH // G)
        def _(g):
            gds = [pltpu.async_copy(table.at[src_b.at[g * G + j, cc]],
                                    rows.at[j, ci], gsem)
                   for j in range(G) for ci, cc in enumerate(cols)]
            for d in gds:
                d.wait()
            sds = [pltpu.async_copy(rows.at[j, ci],
                                    acc.at[dst_b.at[g * G + j, cc]], ssem,
                                    add=True)
                   for j in range(G) for ci, cc in enumerate(cols)]
            for d in sds:
                d.wait()

    plsc.subcore_barrier()
    pltpu.sync_copy(acc.at[pl.ds(base_w, STRW)],
                    part.at[c, pl.ds(base_w, STRW)])


@functools.cache
def _sc_call(cols):
    mesh = plsc.VectorSubcoreMesh(core_axis_name="c", subcore_axis_name="s",
                                  num_cores=NC, num_subcores=NS)
    return pl.kernel(
        functools.partial(_sc_body, cols),
        out_type=jax.ShapeDtypeStruct((NC, ACC_W), _F32),
        mesh=mesh,
        compiler_params=pltpu.CompilerParams(use_tc_tiling_on_sc=False),
        scratch_types=[
            pltpu.VMEM((CH, 4, 128), jnp.int32),
            pltpu.VMEM((CH, 4, 128), jnp.int32),
            pltpu.VMEM((G, len(cols), 128), _F32),
            pltpu.VMEM_SHARED((ACC_W,), _F32),
            pltpu.SemaphoreType.DMA,
            pltpu.SemaphoreType.DMA,
            pltpu.SemaphoreType.DMA,
        ],
    )


def _tca_body(mf_ref, dp_ref, w1_ref, b1_ref, wg1_ref, wg2_ref, woutp_ref,
              u_ref):
    x0 = jnp.dot(mf_ref[...], w1_ref[...],
                 preferred_element_type=_F32) + b1_ref[...][None, :]
    x0 = jnp.where(x0 >= 0, x0, 0.01 * x0)
    kw = jnp.dot(wg2_ref[...], woutp_ref[...], preferred_element_type=_F32)
    kw = jnp.dot(wg1_ref[...], kw, preferred_element_type=_F32)
    zz = jnp.dot(x0, kw, preferred_element_type=_F32)
    deg = dp_ref[:, 2] + 1.0
    dinv = lax.rsqrt(deg)[:, None]
    col = lax.broadcasted_iota(jnp.int32, (BN, WIDTH), 1)
    u_ref[...] = dinv * (zz + jnp.where(col == 2, 1.0, 0.0))


def _tcb_body(pb_ref, u_ref, v_ref):
    u = u_ref[...]
    smat = pb_ref[...] + u
    dinv = u[:, 2:3]
    v = smat * (dinv * dinv)
    r = dinv[:, 0] * smat[:, 2]
    col = lax.broadcasted_iota(jnp.int32, (BN, WIDTH), 1)
    v_ref[...] = jnp.where(col == 3, r[:, None], v)


def _tcc_body(pc_ref, v_ref, u_ref, f_ref, mf_ref, w1_ref, b1_ref, wg2_ref,
              woutp_ref, bg1_ref, bg2_ref, boutp_ref, o_ref):
    v = v_ref[...]
    t = pc_ref[...] + v
    dinv = u_ref[...][:, 2:3]
    r = v[:, 3:4]
    c1 = jnp.dot(jnp.dot(bg1_ref[...][None, :], wg2_ref[...],
                         preferred_element_type=_F32), woutp_ref[...],
                 preferred_element_type=_F32)
    c2 = jnp.dot(bg2_ref[...][None, :], woutp_ref[...],
                 preferred_element_type=_F32) + boutp_ref[...][None, :]
    y = dinv * t + r * c1 + c2
    mk = jnp.dot(f_ref[...] - mf_ref[...], w1_ref[...],
                 preferred_element_type=_F32) + b1_ref[...][None, :]
    mk = jnp.where(mk >= 0, mk, 0.01 * mk)
    mk = jnp.dot(mk, woutp_ref[...],
                 preferred_element_type=_F32) + boutp_ref[...][None, :]
    o_ref[...] = y * mk


def _full(shape):
    return pl.BlockSpec(shape, lambda i: tuple(0 for _ in shape))


_tca_call = pl.pallas_call(
    _tca_body,
    grid=(N_ACC // BN,),
    in_specs=[
        pl.BlockSpec((BN, D_IN), lambda i: (i, 0)),
        pl.BlockSpec((BN, WIDTH), lambda i: (i, 0)),
        _full((D_IN, H)),
        _full((H,)),
        _full((H, H)),
        _full((H, H)),
        _full((H, WIDTH)),
    ],
    out_specs=pl.BlockSpec((BN, WIDTH), lambda i: (i, 0)),
    out_shape=jax.ShapeDtypeStruct((N_ACC, WIDTH), _F32),
)

_tcb_call = pl.pallas_call(
    _tcb_body,
    grid=(N_ACC // BN,),
    in_specs=[
        pl.BlockSpec((BN, WIDTH), lambda i: (i, 0)),
        pl.BlockSpec((BN, WIDTH), lambda i: (i, 0)),
    ],
    out_specs=pl.BlockSpec((BN, WIDTH), lambda i: (i, 0)),
    out_shape=jax.ShapeDtypeStruct((N_ACC, WIDTH), _F32),
)

_tcc_call = pl.pallas_call(
    _tcc_body,
    grid=(N // BNC,),
    in_specs=[
        pl.BlockSpec((BNC, WIDTH), lambda i: (i, 0)),
        pl.BlockSpec((BNC, WIDTH), lambda i: (i, 0)),
        pl.BlockSpec((BNC, WIDTH), lambda i: (i, 0)),
        pl.BlockSpec((BNC, D_IN), lambda i: (i, 0)),
        pl.BlockSpec((BNC, D_IN), lambda i: (i, 0)),
        _full((D_IN, H)),
        _full((H,)),
        _full((H, H)),
        _full((H, WIDTH)),
        _full((H,)),
        _full((H,)),
        _full((WIDTH,)),
    ],
    out_specs=pl.BlockSpec((BNC, WIDTH), lambda i: (i, 0)),
    out_shape=jax.ShapeDtypeStruct((N, WIDTH), _F32),
)


def _word_plane(node_idx):
    r = node_idx.reshape(EROWS, 1, 128) * 4
    return r + jnp.arange(4, dtype=jnp.int32)[None, :, None]


def _merge(p):
    return jnp.concatenate([p[0, :H0 * 4].reshape(H0, WIDTH),
                            p[1, :H0 * 4].reshape(H0, WIDTH)], axis=0)


def kernel(mask_feature, feature, edge_index, edge_type, W1, b1, Wg1, bg1,
           Wg2, bg2, Wout, bout):
    del edge_type
    pad = jnp.full((E_PAD - E,), N, jnp.int32)
    src = jnp.concatenate([edge_index[0], pad])
    dst = jnp.concatenate([edge_index[1], pad])
    loc0 = jnp.where(dst < H0, dst, DUMP)
    loc1 = jnp.where(dst >= H0, dst - H0, DUMP)
    edges = jnp.stack([_word_plane(src), _word_plane(loc0),
                       _word_plane(loc1)])
    mf_p = jnp.zeros((N_ACC, D_IN), _F32).at[:N].set(mask_feature)
    wout_p = jnp.zeros((H, WIDTH), _F32).at[:, :OUT].set(Wout)
    bout_p = jnp.zeros((WIDTH,), _F32).at[:OUT].set(bout)
    ones_t = jnp.ones((N_ACC * 4,), _F32)
    zt = jnp.zeros((128,), _F32)

    deg_call = _sc_call((2,))
    adj_call = _sc_call((0, 1, 2))
    degp = _merge(deg_call(edges, ones_t, zt))
    u = _tca_call(mf_p, degp, W1, b1, Wg1, Wg2, wout_p)
    pb = _merge(adj_call(edges, u.reshape(-1), zt))
    v = _tcb_call(pb, u)
    pc = _merge(adj_call(edges, v.reshape(-1), zt))
    out16 = _tcc_call(pc, v, u, feature, mask_feature, W1, b1, Wg2, wout_p,
                      bg1, bg2, bout_p)
    return out16[:, :OUT]
